# baseline (speedup 1.0000x reference)
import functools

import jax
import jax.numpy as jnp
from jax.experimental import pallas as pl
from jax.experimental.pallas import tpu as pltpu
from jax.experimental.pallas import tpu_sc as plsc

_NB = 256
_CK = 512
_GCHUNK = 128
_NW = 32


def _argmin_body(cb_ref, b_ref, x_ref, out_ref, nk):
    ck = _CK
    x = x_ref[...]
    a = jnp.sum(x * x, axis=1, keepdims=True)

    def step(c, carry):
        run_min, run_idx = carry
        cb = cb_ref[c]
        m = jax.lax.dot_general(x, cb, (((1,), (1,)), ((), ())),
                                preferred_element_type=jnp.float32)
        brow = b_ref[c]
        dist = (a + brow) - 2.0 * m
        cmin = jnp.min(dist, axis=1, keepdims=True)
        iota = jax.lax.broadcasted_iota(jnp.int32, (_NB, ck), 1) + c * ck
        cidx = jnp.min(jnp.where(dist == cmin, iota, jnp.int32(2**30)),
                       axis=1, keepdims=True)
        upd = cmin < run_min
        run_idx = jnp.where(upd, cidx, run_idx)
        run_min = jnp.where(upd, cmin, run_min)
        return run_min, run_idx

    init = (jnp.full((_NB, 1), jnp.inf, jnp.float32),
            jnp.zeros((_NB, 1), jnp.int32))
    _, run_idx = jax.lax.fori_loop(0, nk, step, init)
    out_ref[0] = run_idx


def _argmin_call(codebook, flat):
    n, d = flat.shape
    k = codebook.shape[0]
    nt = n // _NB
    nk = k // _CK
    cb3 = codebook.reshape(nk, _CK, d)
    b3 = jnp.sum(codebook * codebook, axis=1).reshape(nk, 1, _CK)
    out = pl.pallas_call(
        functools.partial(_argmin_body, nk=nk),
        grid=(nt,),
        in_specs=[
            pl.BlockSpec((nk, _CK, d), lambda i: (0, 0, 0)),
            pl.BlockSpec((nk, 1, _CK), lambda i: (0, 0, 0)),
            pl.BlockSpec((_NB, d), lambda i: (i, 0)),
        ],
        out_specs=pl.BlockSpec((1, _NB, 1), lambda i: (i, 0, 0)),
        out_shape=jax.ShapeDtypeStruct((nt, _NB, 1), jnp.int32),
        compiler_params=pltpu.CompilerParams(
            dimension_semantics=("arbitrary",)),
    )(cb3, b3, flat)
    return out


def _sc_gather(codebook, idx_flat):
    n = idx_flat.shape[0]
    d = codebook.shape[1]
    per_w = n // _NW
    nit = per_w // _GCHUNK

    @functools.partial(
        pl.kernel,
        out_type=jax.ShapeDtypeStruct((n, d), jnp.float32),
        mesh=plsc.VectorSubcoreMesh(core_axis_name="c", subcore_axis_name="s"),
        scratch_types=[
            pltpu.VMEM((_GCHUNK,), jnp.int32),
            pltpu.VMEM((_GCHUNK, d), jnp.float32),
            pltpu.SemaphoreType.DMA,
        ],
    )
    def gk(cb_hbm, idx_hbm, out_hbm, idx_v, rows_v, sem):
        wid = jax.lax.axis_index("s") * 2 + jax.lax.axis_index("c")
        base = wid * per_w
        for j in range(nit):
            off = base + j * _GCHUNK
            pltpu.sync_copy(idx_hbm.at[pl.ds(off, _GCHUNK)], idx_v)
            pltpu.async_copy(cb_hbm.at[idx_v], rows_v, sem).wait()
            pltpu.sync_copy(rows_v, out_hbm.at[pl.ds(off, _GCHUNK)])

    return gk(codebook, idx_flat)


def _final_body(z_ref, q_ref, out_ref, s_ref):
    z = z_ref[...]
    q = q_ref[...]
    diff = q - z
    out_ref[...] = z + diff

    @pl.when(pl.program_id(0) == 0)
    def _():
        s_ref[...] = jnp.zeros((1, 1), jnp.float32)

    s_ref[...] += jnp.sum(diff * diff).reshape(1, 1)


def _final_call(flat, q):
    n, d = flat.shape
    nt = n // _NB
    out, s = pl.pallas_call(
        _final_body,
        grid=(nt,),
        in_specs=[
            pl.BlockSpec((_NB, d), lambda i: (i, 0)),
            pl.BlockSpec((_NB, d), lambda i: (i, 0)),
        ],
        out_specs=[
            pl.BlockSpec((_NB, d), lambda i: (i, 0)),
            pl.BlockSpec((1, 1), lambda i: (0, 0)),
        ],
        out_shape=[
            jax.ShapeDtypeStruct((n, d), jnp.float32),
            jax.ShapeDtypeStruct((1, 1), jnp.float32),
        ],
        compiler_params=pltpu.CompilerParams(
            dimension_semantics=("arbitrary",)),
    )(flat, q)
    return out, s


def kernel(z, codebook):
    b, t, d = z.shape
    flat = z.reshape(-1, d)
    idx3 = _argmin_call(codebook, flat)
    idx_flat = idx3.reshape(-1)
    q = _sc_gather(codebook, idx_flat)
    qst, s = _final_call(flat, q)
    m = s[0, 0] / jnp.float32(flat.size)
    loss = m + jnp.float32(0.25) * m
    return qst.reshape(b, t, d), loss, idx3.reshape(b, t)
